# baseline (speedup 1.0000x reference)
import functools

import jax
import jax.numpy as jnp
from jax import lax
from jax.experimental import pallas as pl
from jax.experimental.pallas import tpu as pltpu
from jax.experimental.pallas import tpu_sc as plsc

H = 4096
W = 4096
N = 4194304
NPIX = H * W
NROW = NPIX // 4

_NC = 2
_NS = 16
_NW = _NC * _NS
_NPW = N // _NW
_C = 2048
_NIT = _NPW // _C
_VR = _C // 16

_BUILD_LO = W // 4
_BUILD_COUNT = NROW - 2 * _BUILD_LO
_RPW = _BUILD_COUNT // _NW

_BB = 2048
_BNB = _RPW // _BB
_BREM = _RPW - _BNB * _BB

_OXW = 0
_OYM = 4 * _BB + 16
_OYP = 8 * _BB + 16
_SB = 12 * _BB + 16


def _build_body(e_hbm, t_hbm, s0_v, s1_v, t_v, sem0, sem1):
    wid = lax.axis_index("s") * _NC + lax.axis_index("c")
    wlo = wid * _RPW
    iota16 = lax.iota(jnp.int32, 16)
    svs = (s0_v, s1_v)
    sems = (sem0, sem1)
    pat = jnp.where(iota16 < 6, 7 + iota16,
                    jnp.where(iota16 < 10, _OYM + (iota16 - 6),
                              jnp.where(iota16 < 14, _OYP + (iota16 - 10),
                                        0)))

    def _copies(m, nrows, slot, mk):
        g = 4 * (_BUILD_LO + m)
        sv = svs[slot]
        return [
            mk(e_hbm.at[pl.ds(g - 8, 4 * nrows + 16)],
               sv.at[pl.ds(_OXW, 4 * nrows + 16)], sems[slot]),
            mk(e_hbm.at[pl.ds(g - W, 4 * nrows)],
               sv.at[pl.ds(_OYM, 4 * nrows)], sems[slot]),
            mk(e_hbm.at[pl.ds(g + W, 4 * nrows)],
               sv.at[pl.ds(_OYP, 4 * nrows)], sems[slot]),
        ]

    def stage(m, nrows, slot):
        _copies(m, nrows, slot, pltpu.async_copy)

    def assemble(m, nrows, slot):
        for cp in _copies(m, nrows, slot, pltpu.make_async_copy):
            cp.wait()
        sv = svs[slot]

        def grp(v, c2):
            for u in range(2):
                l = 2 * v + u
                val = plsc.load_gather(sv, [pat + 4 * l])
                plsc.store_scatter(t_v, [jnp.full((16,), 0, jnp.int32) + l,
                                         iota16], val)
            return c2

        lax.fori_loop(0, nrows // 2, grp, None)
        pltpu.sync_copy(t_v.at[pl.ds(0, nrows)] if nrows != _BB else t_v,
                        t_hbm.at[pl.ds(_BUILD_LO + m, nrows)])

    stage(wlo, _BB, 0)

    def outer(k, c):
        b = 2 * k
        stage(wlo + (b + 1) * _BB, _BB, 1)
        assemble(wlo + b * _BB, _BB, 0)
        stage(wlo + (b + 2) * _BB, _BB, 0)
        assemble(wlo + (b + 1) * _BB, _BB, 1)
        return c

    lax.fori_loop(0, (_BNB - 1) // 2, outer, None)
    stage(wlo + _BNB * _BB, _BREM, 1)
    assemble(wlo + (_BNB - 1) * _BB, _BB, 0)
    assemble(wlo + _BNB * _BB, _BREM, 1)


_sc_build = functools.partial(
    pl.kernel,
    mesh=plsc.VectorSubcoreMesh(core_axis_name="c", subcore_axis_name="s"),
    out_type=jax.ShapeDtypeStruct((NROW, 16), jnp.float32),
    scratch_types=[
        pltpu.VMEM((_SB,), jnp.float32),
        pltpu.VMEM((_SB,), jnp.float32),
        pltpu.VMEM((_BB, 16), jnp.float32),
        pltpu.SemaphoreType.DMA,
        pltpu.SemaphoreType.DMA,
    ],
    compiler_params=pltpu.CompilerParams(use_tc_tiling_on_sc=False,
                                         needs_layout_passes=False),
)(_build_body)


def _gather_body(xx_hbm, xy_hbm, t_hbm, out_hbm,
                 xs_v, ys_v, idx0_v, idx1_v, g0_v, g1_v, o_v,
                 sem0, sem1):
    wid = lax.axis_index("s") * _NC + lax.axis_index("c")
    wbase = wid * _NPW
    sems = (sem0, sem1)
    idxs = (idx0_v, idx1_v)
    gs = (g0_v, g1_v)
    iota16 = lax.iota(jnp.int32, 16)

    def fire(i, slot):
        base = wbase + i * _C
        xsb, ysb, idxb = xs_v.at[slot], ys_v.at[slot], idxs[slot]
        pltpu.sync_copy(xx_hbm.at[pl.ds(base, _C)], xsb)
        pltpu.sync_copy(xy_hbm.at[pl.ds(base, _C)], ysb)

        def build(j, c):
            lane = j * 16
            sx = xsb[pl.ds(lane, 16)] * 2048.0 + 2048.0
            sy = ysb[pl.ds(lane, 16)] * 2048.0 + 2048.0
            ixc = jnp.clip(sx.astype(jnp.int32), 1, W - 2)
            iyc = jnp.clip(sy.astype(jnp.int32), 1, H - 2)
            flat = iyc * W + ixc
            idxb[pl.ds(lane, 16)] = lax.shift_right_logical(flat, 2)
            return c

        lax.fori_loop(0, _VR, build, None)
        pltpu.async_copy(t_hbm.at[idxb], gs[slot], sems[slot])

    def drain(i, slot):
        base = wbase + i * _C
        xsb, ysb = xs_v.at[slot], ys_v.at[slot]
        gb = gs[slot]
        pltpu.make_async_copy(t_hbm.at[idxs[slot]], gb, sems[slot]).wait()

        def combine(j, c):
            lane = j * 16
            sx = xsb[pl.ds(lane, 16)] * 2048.0 + 2048.0
            sy = ysb[pl.ds(lane, 16)] * 2048.0 + 2048.0
            ix = sx.astype(jnp.int32)
            iy = sy.astype(jnp.int32)
            fx = sx - ix.astype(jnp.float32)
            fy = sy - iy.astype(jnp.float32)
            ixc = jnp.clip(ix, 1, W - 2)
            iyc = jnp.clip(iy, 1, H - 2)
            flat = iyc * W + ixc
            jj = jnp.bitwise_and(flat, 3)
            pt = lane + iota16
            exm = plsc.load_gather(gb, [pt, jj])
            e0 = plsc.load_gather(gb, [pt, jj + 1])
            exp_ = plsc.load_gather(gb, [pt, jj + 2])
            eym = plsc.load_gather(gb, [pt, jj + 6])
            eyp = plsc.load_gather(gb, [pt, jj + 10])
            dedx = 0.5 * (exp_ - exm)
            dedy = 0.5 * (eyp - eym)
            zero = jnp.float32(0.0)
            dx = jnp.maximum(jnp.maximum(-sx, zero),
                             jnp.maximum(sx - (W - 1), zero)) * (1.0 / 2048.0)
            dy = jnp.maximum(jnp.maximum(-sy, zero),
                             jnp.maximum(sy - (H - 1), zero)) * (1.0 / 2048.0)
            pen = dx * dx + dy * dy
            grad = fx * dedx + fy * dedy
            o_v[pl.ds(lane, 16)] = e0 + jnp.where(pen < 1e-6, grad, zero) + pen
            return c

        lax.fori_loop(0, _VR, combine, None)
        pltpu.sync_copy(o_v, out_hbm.at[pl.ds(base, _C)])

    fire(0, 0)

    def outer(k, carry):
        i = 2 * k
        fire(i + 1, 1)
        drain(i, 0)
        fire(i + 2, 0)
        drain(i + 1, 1)
        return carry

    lax.fori_loop(0, _NIT // 2 - 1, outer, None)
    fire(_NIT - 1, 1)
    drain(_NIT - 2, 0)
    drain(_NIT - 1, 1)


_sc_gather = functools.partial(
    pl.kernel,
    mesh=plsc.VectorSubcoreMesh(core_axis_name="c", subcore_axis_name="s"),
    out_type=jax.ShapeDtypeStruct((N,), jnp.float32),
    scratch_types=[
        pltpu.VMEM((2, _C), jnp.float32),
        pltpu.VMEM((2, _C), jnp.float32),
        pltpu.VMEM((_C,), jnp.int32),
        pltpu.VMEM((_C,), jnp.int32),
        pltpu.VMEM((_C, 16), jnp.float32),
        pltpu.VMEM((_C, 16), jnp.float32),
        pltpu.VMEM((_C,), jnp.float32),
        pltpu.SemaphoreType.DMA,
        pltpu.SemaphoreType.DMA,
    ],
    compiler_params=pltpu.CompilerParams(use_tc_tiling_on_sc=False,
                                         needs_layout_passes=False),
)(_gather_body)


def kernel(X, pixel_energy):
    e = pixel_energy.reshape(-1)
    t = _sc_build(e)
    out = _sc_gather(X[:, 0], X[:, 1], t)
    return out[:, None]


# speedup vs baseline: 1.0160x; 1.0160x over previous
import functools

import jax
import jax.numpy as jnp
from jax import lax
from jax.experimental import pallas as pl
from jax.experimental.pallas import tpu as pltpu
from jax.experimental.pallas import tpu_sc as plsc

H = 4096
W = 4096
N = 4194304
NPIX = H * W
NROW = NPIX // 2

_NC = 2
_NS = 16
_NW = _NC * _NS
_NPW = N // _NW
_C = 4096
_NIT = _NPW // _C
_VR = _C // 16

_BUILD_LO = W // 2
_BUILD_COUNT = NROW - 2 * _BUILD_LO
_RPW = _BUILD_COUNT // _NW

_BB = 2048
_BNB = _RPW // _BB
_BREM = _RPW - _BNB * _BB

_OXW = 0
_OYM = 2 * _BB + 16
_OYP = 4 * _BB + 16
_SB = 6 * _BB + 16


def _build_body(e_hbm, t_hbm, s0_v, s1_v, t_v, sem0, sem1):
    wid = lax.axis_index("s") * _NC + lax.axis_index("c")
    wlo = wid * _RPW
    iota16 = lax.iota(jnp.int32, 16)
    svs = (s0_v, s1_v)
    sems = (sem0, sem1)
    rowpat = lax.shift_right_logical(iota16, 3)
    colpat = jnp.bitwise_and(iota16, 7)
    pat = jnp.where(colpat < 4, 7 + colpat,
                    jnp.where(colpat < 6, _OYM + (colpat - 4),
                              _OYP + (colpat - 6))) + 2 * rowpat

    def _copies(m, nrows, slot, mk):
        g = 2 * (_BUILD_LO + m)
        sv = svs[slot]
        return [
            mk(e_hbm.at[pl.ds(g - 8, 2 * nrows + 16)],
               sv.at[pl.ds(_OXW, 2 * nrows + 16)], sems[slot]),
            mk(e_hbm.at[pl.ds(g - W, 2 * nrows)],
               sv.at[pl.ds(_OYM, 2 * nrows)], sems[slot]),
            mk(e_hbm.at[pl.ds(g + W, 2 * nrows)],
               sv.at[pl.ds(_OYP, 2 * nrows)], sems[slot]),
        ]

    def stage(m, nrows, slot):
        _copies(m, nrows, slot, pltpu.async_copy)

    def assemble(m, nrows, slot):
        for cp in _copies(m, nrows, slot, pltpu.make_async_copy):
            cp.wait()
        sv = svs[slot]

        def grp(v, c2):
            for u in range(4):
                l0 = (4 * v + u) * 2
                idx = pat + 4 * (4 * v + u)
                val = plsc.load_gather(sv, [idx])
                plsc.store_scatter(t_v, [rowpat + l0, colpat], val)
            return c2

        lax.fori_loop(0, nrows // 8, grp, None)
        pltpu.sync_copy(t_v.at[pl.ds(0, nrows)] if nrows != _BB else t_v,
                        t_hbm.at[pl.ds(_BUILD_LO + m, nrows)])

    stage(wlo, _BB, 0)

    def outer(k, c):
        b = 2 * k
        stage(wlo + (b + 1) * _BB, _BB, 1)
        assemble(wlo + b * _BB, _BB, 0)
        stage(wlo + (b + 2) * _BB, _BB, 0)
        assemble(wlo + (b + 1) * _BB, _BB, 1)
        return c

    lax.fori_loop(0, (_BNB - 1) // 2, outer, None)
    stage(wlo + _BNB * _BB, _BREM, 1)
    assemble(wlo + (_BNB - 1) * _BB, _BB, 0)
    assemble(wlo + _BNB * _BB, _BREM, 1)


_sc_build = functools.partial(
    pl.kernel,
    mesh=plsc.VectorSubcoreMesh(core_axis_name="c", subcore_axis_name="s"),
    out_type=jax.ShapeDtypeStruct((NROW, 8), jnp.float32),
    scratch_types=[
        pltpu.VMEM((_SB,), jnp.float32),
        pltpu.VMEM((_SB,), jnp.float32),
        pltpu.VMEM((_BB, 8), jnp.float32),
        pltpu.SemaphoreType.DMA,
        pltpu.SemaphoreType.DMA,
    ],
    compiler_params=pltpu.CompilerParams(use_tc_tiling_on_sc=False,
                                         needs_layout_passes=False),
)(_build_body)


def _gather_body(xx_hbm, xy_hbm, t_hbm, out_hbm,
                 xs_v, ys_v, idx0_v, idx1_v, g0_v, g1_v, o_v,
                 sem0, sem1):
    wid = lax.axis_index("s") * _NC + lax.axis_index("c")
    wbase = wid * _NPW
    sems = (sem0, sem1)
    idxs = (idx0_v, idx1_v)
    gs = (g0_v, g1_v)
    iota16 = lax.iota(jnp.int32, 16)
    iota2 = 2 * iota16

    def fire(i, slot):
        base = wbase + i * _C
        xsb, ysb, idxb = xs_v.at[slot], ys_v.at[slot], idxs[slot]
        pltpu.sync_copy(xx_hbm.at[pl.ds(base, _C)], xsb)
        pltpu.sync_copy(xy_hbm.at[pl.ds(base, _C)], ysb)

        def build(j, c):
            lane = j * 16
            sx = xsb[pl.ds(lane, 16)] * 2048.0 + 2048.0
            sy = ysb[pl.ds(lane, 16)] * 2048.0 + 2048.0
            ixc = jnp.clip(sx.astype(jnp.int32), 1, W - 2)
            iyc = jnp.clip(sy.astype(jnp.int32), 1, H - 2)
            flat = iyc * W + ixc
            idxb[pl.ds(lane, 16)] = lax.shift_right_logical(flat, 1)
            return c

        lax.fori_loop(0, _VR, build, None)
        pltpu.async_copy(t_hbm.at[idxb], gs[slot], sems[slot])

    def drain(i, slot):
        base = wbase + i * _C
        xsb, ysb = xs_v.at[slot], ys_v.at[slot]
        gb = gs[slot]
        pltpu.make_async_copy(t_hbm.at[idxs[slot]], gb, sems[slot]).wait()

        def combine(j, c):
            lane = j * 16
            sx = xsb[pl.ds(lane, 16)] * 2048.0 + 2048.0
            sy = ysb[pl.ds(lane, 16)] * 2048.0 + 2048.0
            ix = sx.astype(jnp.int32)
            iy = sy.astype(jnp.int32)
            fx = sx - ix.astype(jnp.float32)
            fy = sy - iy.astype(jnp.float32)
            ixc = jnp.clip(ix, 1, W - 2)
            iyc = jnp.clip(iy, 1, H - 2)
            flat = iyc * W + ixc
            jj = jnp.bitwise_and(flat, 1)
            pt = lane + iota16
            exm = plsc.load_gather(gb, [pt, jj])
            e0 = plsc.load_gather(gb, [pt, jj + 1])
            exp_ = plsc.load_gather(gb, [pt, jj + 2])
            eym = plsc.load_gather(gb, [pt, jj + 4])
            eyp = plsc.load_gather(gb, [pt, jj + 6])
            dedx = 0.5 * (exp_ - exm)
            dedy = 0.5 * (eyp - eym)
            zero = jnp.float32(0.0)
            dx = jnp.maximum(jnp.maximum(-sx, zero),
                             jnp.maximum(sx - (W - 1), zero)) * (1.0 / 2048.0)
            dy = jnp.maximum(jnp.maximum(-sy, zero),
                             jnp.maximum(sy - (H - 1), zero)) * (1.0 / 2048.0)
            pen = dx * dx + dy * dy
            grad = fx * dedx + fy * dedy
            o_v[pl.ds(lane, 16)] = e0 + jnp.where(pen < 1e-6, grad, zero) + pen
            return c

        lax.fori_loop(0, _VR, combine, None)
        pltpu.sync_copy(o_v, out_hbm.at[pl.ds(base, _C)])

    fire(0, 0)

    def outer(k, carry):
        i = 2 * k
        fire(i + 1, 1)
        drain(i, 0)
        fire(i + 2, 0)
        drain(i + 1, 1)
        return carry

    lax.fori_loop(0, _NIT // 2 - 1, outer, None)
    fire(_NIT - 1, 1)
    drain(_NIT - 2, 0)
    drain(_NIT - 1, 1)


_sc_gather = functools.partial(
    pl.kernel,
    mesh=plsc.VectorSubcoreMesh(core_axis_name="c", subcore_axis_name="s"),
    out_type=jax.ShapeDtypeStruct((N,), jnp.float32),
    scratch_types=[
        pltpu.VMEM((2, _C), jnp.float32),
        pltpu.VMEM((2, _C), jnp.float32),
        pltpu.VMEM((_C,), jnp.int32),
        pltpu.VMEM((_C,), jnp.int32),
        pltpu.VMEM((_C, 8), jnp.float32),
        pltpu.VMEM((_C, 8), jnp.float32),
        pltpu.VMEM((_C,), jnp.float32),
        pltpu.SemaphoreType.DMA,
        pltpu.SemaphoreType.DMA,
    ],
    compiler_params=pltpu.CompilerParams(use_tc_tiling_on_sc=False,
                                         needs_layout_passes=False),
)(_gather_body)


def kernel(X, pixel_energy):
    e = pixel_energy.reshape(-1)
    t = _sc_build(e)
    sel = jnp.array([[1.0, 0.0], [0.0, 1.0]], jnp.float32)
    xx = jnp.sum(X * sel[0], axis=1)
    xy = jnp.sum(X * sel[1], axis=1)
    out = _sc_gather(xx, xy, t)
    return out[:, None]


# speedup vs baseline: 1.0747x; 1.0579x over previous
import functools

import jax
import jax.numpy as jnp
from jax import lax
from jax.experimental import pallas as pl
from jax.experimental.pallas import tpu as pltpu
from jax.experimental.pallas import tpu_sc as plsc

H = 4096
W = 4096
N = 4194304
NPIX = H * W
NROW = NPIX // 2

_NC = 2
_NS = 16
_NW = _NC * _NS
_NPW = N // _NW
_C = 4096
_NIT = _NPW // _C
_VR = _C // 16

_BUILD_LO = W // 2
_BUILD_COUNT = NROW - 2 * _BUILD_LO
_RPW = _BUILD_COUNT // _NW

_BB = 2048
_BNB = _RPW // _BB
_BREM = _RPW - _BNB * _BB

_OXW = 0
_OYM = 2 * _BB + 16
_OYP = 4 * _BB + 16
_SB = 6 * _BB + 16


def _build_body(e_hbm, t_hbm, s0_v, s1_v, t_v, sem0, sem1):
    wid = lax.axis_index("s") * _NC + lax.axis_index("c")
    wlo = wid * _RPW
    iota16 = lax.iota(jnp.int32, 16)
    svs = (s0_v, s1_v)
    sems = (sem0, sem1)
    rowpat = lax.shift_right_logical(iota16, 3)
    colpat = jnp.bitwise_and(iota16, 7)
    pat = jnp.where(colpat < 4, 7 + colpat,
                    jnp.where(colpat < 6, _OYM + (colpat - 4),
                              _OYP + (colpat - 6))) + 2 * rowpat

    def _copies(m, nrows, slot, mk):
        g = 2 * (_BUILD_LO + m)
        sv = svs[slot]
        return [
            mk(e_hbm.at[pl.ds(g - 8, 2 * nrows + 16)],
               sv.at[pl.ds(_OXW, 2 * nrows + 16)], sems[slot]),
            mk(e_hbm.at[pl.ds(g - W, 2 * nrows)],
               sv.at[pl.ds(_OYM, 2 * nrows)], sems[slot]),
            mk(e_hbm.at[pl.ds(g + W, 2 * nrows)],
               sv.at[pl.ds(_OYP, 2 * nrows)], sems[slot]),
        ]

    def stage(m, nrows, slot):
        _copies(m, nrows, slot, pltpu.async_copy)

    def assemble(m, nrows, slot):
        for cp in _copies(m, nrows, slot, pltpu.make_async_copy):
            cp.wait()
        sv = svs[slot]

        def grp(v, c2):
            for u in range(4):
                l0 = (4 * v + u) * 2
                idx = pat + 4 * (4 * v + u)
                val = plsc.load_gather(sv, [idx])
                plsc.store_scatter(t_v, [rowpat + l0, colpat], val)
            return c2

        lax.fori_loop(0, nrows // 8, grp, None)
        pltpu.sync_copy(t_v.at[pl.ds(0, nrows)] if nrows != _BB else t_v,
                        t_hbm.at[pl.ds(_BUILD_LO + m, nrows)])

    stage(wlo, _BB, 0)

    def outer(k, c):
        b = 2 * k
        stage(wlo + (b + 1) * _BB, _BB, 1)
        assemble(wlo + b * _BB, _BB, 0)
        stage(wlo + (b + 2) * _BB, _BB, 0)
        assemble(wlo + (b + 1) * _BB, _BB, 1)
        return c

    lax.fori_loop(0, (_BNB - 1) // 2, outer, None)
    stage(wlo + _BNB * _BB, _BREM, 1)
    assemble(wlo + (_BNB - 1) * _BB, _BB, 0)
    assemble(wlo + _BNB * _BB, _BREM, 1)


_sc_build = functools.partial(
    pl.kernel,
    mesh=plsc.VectorSubcoreMesh(core_axis_name="c", subcore_axis_name="s"),
    out_type=jax.ShapeDtypeStruct((NROW, 8), jnp.float32),
    scratch_types=[
        pltpu.VMEM((_SB,), jnp.float32),
        pltpu.VMEM((_SB,), jnp.float32),
        pltpu.VMEM((_BB, 8), jnp.float32),
        pltpu.SemaphoreType.DMA,
        pltpu.SemaphoreType.DMA,
    ],
    compiler_params=pltpu.CompilerParams(use_tc_tiling_on_sc=False,
                                         needs_layout_passes=False),
)(_build_body)


def _gather_body(xx_hbm, xy_hbm, t_hbm, out_hbm,
                 xs_v, ys_v, idx0_v, idx1_v, g0_v, g1_v, o_v,
                 sem0, sem1):
    wid = lax.axis_index("s") * _NC + lax.axis_index("c")
    wbase = wid * _NPW
    sems = (sem0, sem1)
    idxs = (idx0_v, idx1_v)
    gs = (g0_v, g1_v)
    iota16 = lax.iota(jnp.int32, 16)
    iota2 = 2 * iota16

    def fire(i, slot):
        base = wbase + i * _C
        xsb, ysb, idxb = xs_v.at[slot], ys_v.at[slot], idxs[slot]
        pltpu.sync_copy(xx_hbm.at[pl.ds(base, _C)], xsb)
        pltpu.sync_copy(xy_hbm.at[pl.ds(base, _C)], ysb)

        def build(j, c):
            lane = j * 16
            sx = xsb[pl.ds(lane, 16)] * 2048.0 + 2048.0
            sy = ysb[pl.ds(lane, 16)] * 2048.0 + 2048.0
            ixc = jnp.clip(sx.astype(jnp.int32), 1, W - 2)
            iyc = jnp.clip(sy.astype(jnp.int32), 1, H - 2)
            flat = iyc * W + ixc
            idxb[pl.ds(lane, 16)] = lax.shift_right_logical(flat, 1)
            return c

        lax.fori_loop(0, _VR, build, None)
        pltpu.async_copy(t_hbm.at[idxb], gs[slot], sems[slot])

    def drain(i, slot):
        base = wbase + i * _C
        xsb, ysb = xs_v.at[slot], ys_v.at[slot]
        gb = gs[slot]
        pltpu.make_async_copy(t_hbm.at[idxs[slot]], gb, sems[slot]).wait()

        def combine(j, c):
            lane = j * 16
            sx = xsb[pl.ds(lane, 16)] * 2048.0 + 2048.0
            sy = ysb[pl.ds(lane, 16)] * 2048.0 + 2048.0
            ix = sx.astype(jnp.int32)
            iy = sy.astype(jnp.int32)
            fx = sx - ix.astype(jnp.float32)
            fy = sy - iy.astype(jnp.float32)
            ixc = jnp.clip(ix, 1, W - 2)
            iyc = jnp.clip(iy, 1, H - 2)
            flat = iyc * W + ixc
            jj = jnp.bitwise_and(flat, 1)
            pt = lane + iota16
            exm = plsc.load_gather(gb, [pt, jj])
            e0 = plsc.load_gather(gb, [pt, jj + 1])
            exp_ = plsc.load_gather(gb, [pt, jj + 2])
            eym = plsc.load_gather(gb, [pt, jj + 4])
            eyp = plsc.load_gather(gb, [pt, jj + 6])
            dedx = 0.5 * (exp_ - exm)
            dedy = 0.5 * (eyp - eym)
            zero = jnp.float32(0.0)
            dx = jnp.maximum(jnp.maximum(-sx, zero),
                             jnp.maximum(sx - (W - 1), zero)) * (1.0 / 2048.0)
            dy = jnp.maximum(jnp.maximum(-sy, zero),
                             jnp.maximum(sy - (H - 1), zero)) * (1.0 / 2048.0)
            pen = dx * dx + dy * dy
            grad = fx * dedx + fy * dedy
            o_v[pl.ds(lane, 16)] = e0 + jnp.where(pen < 1e-6, grad, zero) + pen
            return c

        lax.fori_loop(0, _VR, combine, None)
        pltpu.sync_copy(o_v, out_hbm.at[pl.ds(base, _C)])

    fire(0, 0)

    def outer(k, carry):
        i = 2 * k
        fire(i + 1, 1)
        drain(i, 0)
        fire(i + 2, 0)
        drain(i + 1, 1)
        return carry

    lax.fori_loop(0, _NIT // 2 - 1, outer, None)
    fire(_NIT - 1, 1)
    drain(_NIT - 2, 0)
    drain(_NIT - 1, 1)


_sc_gather = functools.partial(
    pl.kernel,
    mesh=plsc.VectorSubcoreMesh(core_axis_name="c", subcore_axis_name="s"),
    out_type=jax.ShapeDtypeStruct((N,), jnp.float32),
    scratch_types=[
        pltpu.VMEM((2, _C), jnp.float32),
        pltpu.VMEM((2, _C), jnp.float32),
        pltpu.VMEM((_C,), jnp.int32),
        pltpu.VMEM((_C,), jnp.int32),
        pltpu.VMEM((_C, 8), jnp.float32),
        pltpu.VMEM((_C, 8), jnp.float32),
        pltpu.VMEM((_C,), jnp.float32),
        pltpu.SemaphoreType.DMA,
        pltpu.SemaphoreType.DMA,
    ],
    compiler_params=pltpu.CompilerParams(use_tc_tiling_on_sc=False,
                                         needs_layout_passes=False),
)(_gather_body)


def kernel(X, pixel_energy):
    e = pixel_energy.reshape(-1)
    t = _sc_build(e)
    out = _sc_gather(X[:, 0], X[:, 1], t)
    return out[:, None]


# speedup vs baseline: 1.1622x; 1.0814x over previous
import functools

import jax
import jax.numpy as jnp
from jax import lax
from jax.experimental import pallas as pl
from jax.experimental.pallas import tpu as pltpu
from jax.experimental.pallas import tpu_sc as plsc

H = 4096
W = 4096
N = 4194304
NPIX = H * W
NROW = NPIX // 2

_NC = 2
_NS = 16
_NW = _NC * _NS
_NPW = N // _NW
_C = 4096
_NIT = _NPW // _C
_VR = _C // 16

_BUILD_LO = W // 2
_BUILD_COUNT = NROW - 2 * _BUILD_LO
_RPW = _BUILD_COUNT // _NW

_BB = 2048
_BNB = _RPW // _BB
_BREM = _RPW - _BNB * _BB

_OXW = 0
_OYM = 2 * _BB + 16
_OYP = 4 * _BB + 16
_SB = 6 * _BB + 16


def _build_body(e_hbm, t_hbm, s0_v, s1_v, t0_v, t1_v,
                sem0, sem1, wsem0, wsem1):
    wid = lax.axis_index("s") * _NC + lax.axis_index("c")
    wlo = wid * _RPW
    iota16 = lax.iota(jnp.int32, 16)
    svs = (s0_v, s1_v)
    tvs = (t0_v, t1_v)
    sems = (sem0, sem1)
    wsems = (wsem0, wsem1)
    rowpat = lax.shift_right_logical(iota16, 3)
    colpat = jnp.bitwise_and(iota16, 7)
    pat = jnp.where(colpat < 4, 7 + colpat,
                    jnp.where(colpat < 6, _OYM + (colpat - 4),
                              _OYP + (colpat - 6))) + 2 * rowpat

    def _copies(m, nrows, slot, mk):
        g = 2 * (_BUILD_LO + m)
        sv = svs[slot]
        return [
            mk(e_hbm.at[pl.ds(g - 8, 2 * nrows + 16)],
               sv.at[pl.ds(_OXW, 2 * nrows + 16)], sems[slot]),
            mk(e_hbm.at[pl.ds(g - W, 2 * nrows)],
               sv.at[pl.ds(_OYM, 2 * nrows)], sems[slot]),
            mk(e_hbm.at[pl.ds(g + W, 2 * nrows)],
               sv.at[pl.ds(_OYP, 2 * nrows)], sems[slot]),
        ]

    def stage(m, nrows, slot):
        _copies(m, nrows, slot, pltpu.async_copy)

    def _wb(m, nrows, slot, mk):
        tv = tvs[slot]
        return mk(tv.at[pl.ds(0, nrows)] if nrows != _BB else tv,
                  t_hbm.at[pl.ds(_BUILD_LO + m, nrows)], wsems[slot])

    def assemble(m, nrows, slot, wait_prev):
        for cp in _copies(m, nrows, slot, pltpu.make_async_copy):
            cp.wait()
        if wait_prev:
            _wb(m, _BB, slot, pltpu.make_async_copy).wait()
        sv = svs[slot]
        tv = tvs[slot]

        def grp(v, c2):
            for u in range(2):
                l0 = (2 * v + u) * 2
                idx = pat + 4 * (2 * v + u)
                val = plsc.load_gather(sv, [idx])
                plsc.store_scatter(tv, [rowpat + l0, colpat], val)
            return c2

        lax.fori_loop(0, nrows // 4, grp, None)
        _wb(m, nrows, slot, pltpu.async_copy)

    stage(wlo, _BB, 0)
    stage(wlo + _BB, _BB, 1)
    assemble(wlo, _BB, 0, False)
    stage(wlo + 2 * _BB, _BB, 0)
    assemble(wlo + _BB, _BB, 1, False)

    def outer(k, c):
        b = 2 * k + 2
        stage(wlo + (b + 1) * _BB, _BB, 1)
        assemble(wlo + b * _BB, _BB, 0, True)
        stage(wlo + (b + 2) * _BB, _BB, 0)
        assemble(wlo + (b + 1) * _BB, _BB, 1, True)
        return c

    lax.fori_loop(0, (_BNB - 3) // 2, outer, None)
    stage(wlo + _BNB * _BB, _BREM, 1)
    assemble(wlo + (_BNB - 1) * _BB, _BB, 0, True)
    assemble(wlo + _BNB * _BB, _BREM, 1, True)
    _wb(wlo + (_BNB - 1) * _BB, _BB, 0, pltpu.make_async_copy).wait()
    _wb(wlo + _BNB * _BB, _BREM, 1, pltpu.make_async_copy).wait()


_sc_build = functools.partial(
    pl.kernel,
    mesh=plsc.VectorSubcoreMesh(core_axis_name="c", subcore_axis_name="s"),
    out_type=jax.ShapeDtypeStruct((NROW, 8), jnp.float32),
    scratch_types=[
        pltpu.VMEM((_SB,), jnp.float32),
        pltpu.VMEM((_SB,), jnp.float32),
        pltpu.VMEM((_BB, 8), jnp.float32),
        pltpu.VMEM((_BB, 8), jnp.float32),
        pltpu.SemaphoreType.DMA,
        pltpu.SemaphoreType.DMA,
        pltpu.SemaphoreType.DMA,
        pltpu.SemaphoreType.DMA,
    ],
    compiler_params=pltpu.CompilerParams(use_tc_tiling_on_sc=False,
                                         needs_layout_passes=False),
)(_build_body)


def _gather_body(xx_hbm, xy_hbm, t_hbm, out_hbm,
                 xs_v, ys_v, idx0_v, idx1_v, g0_v, g1_v, o_v,
                 sem0, sem1):
    wid = lax.axis_index("s") * _NC + lax.axis_index("c")
    wbase = wid * _NPW
    sems = (sem0, sem1)
    idxs = (idx0_v, idx1_v)
    gs = (g0_v, g1_v)
    iota16 = lax.iota(jnp.int32, 16)
    iota2 = 2 * iota16

    def fire(i, slot):
        base = wbase + i * _C
        xsb, ysb, idxb = xs_v.at[slot], ys_v.at[slot], idxs[slot]
        pltpu.sync_copy(xx_hbm.at[pl.ds(base, _C)], xsb)
        pltpu.sync_copy(xy_hbm.at[pl.ds(base, _C)], ysb)

        def build(j, c):
            lane = j * 16
            sx = xsb[pl.ds(lane, 16)] * 2048.0 + 2048.0
            sy = ysb[pl.ds(lane, 16)] * 2048.0 + 2048.0
            ixc = jnp.clip(sx.astype(jnp.int32), 1, W - 2)
            iyc = jnp.clip(sy.astype(jnp.int32), 1, H - 2)
            flat = iyc * W + ixc
            idxb[pl.ds(lane, 16)] = lax.shift_right_logical(flat, 1)
            return c

        lax.fori_loop(0, _VR, build, None)
        pltpu.async_copy(t_hbm.at[idxb], gs[slot], sems[slot])

    def drain(i, slot):
        base = wbase + i * _C
        xsb, ysb = xs_v.at[slot], ys_v.at[slot]
        gb = gs[slot]
        pltpu.make_async_copy(t_hbm.at[idxs[slot]], gb, sems[slot]).wait()

        def combine(j, c):
            lane = j * 16
            sx = xsb[pl.ds(lane, 16)] * 2048.0 + 2048.0
            sy = ysb[pl.ds(lane, 16)] * 2048.0 + 2048.0
            ix = sx.astype(jnp.int32)
            iy = sy.astype(jnp.int32)
            fx = sx - ix.astype(jnp.float32)
            fy = sy - iy.astype(jnp.float32)
            ixc = jnp.clip(ix, 1, W - 2)
            iyc = jnp.clip(iy, 1, H - 2)
            flat = iyc * W + ixc
            jj = jnp.bitwise_and(flat, 1)
            pt = lane + iota16
            exm = plsc.load_gather(gb, [pt, jj])
            e0 = plsc.load_gather(gb, [pt, jj + 1])
            exp_ = plsc.load_gather(gb, [pt, jj + 2])
            eym = plsc.load_gather(gb, [pt, jj + 4])
            eyp = plsc.load_gather(gb, [pt, jj + 6])
            dedx = 0.5 * (exp_ - exm)
            dedy = 0.5 * (eyp - eym)
            zero = jnp.float32(0.0)
            dx = jnp.maximum(jnp.maximum(-sx, zero),
                             jnp.maximum(sx - (W - 1), zero)) * (1.0 / 2048.0)
            dy = jnp.maximum(jnp.maximum(-sy, zero),
                             jnp.maximum(sy - (H - 1), zero)) * (1.0 / 2048.0)
            pen = dx * dx + dy * dy
            grad = fx * dedx + fy * dedy
            o_v[pl.ds(lane, 16)] = e0 + jnp.where(pen < 1e-6, grad, zero) + pen
            return c

        lax.fori_loop(0, _VR, combine, None)
        pltpu.sync_copy(o_v, out_hbm.at[pl.ds(base, _C)])

    fire(0, 0)

    def outer(k, carry):
        i = 2 * k
        fire(i + 1, 1)
        drain(i, 0)
        fire(i + 2, 0)
        drain(i + 1, 1)
        return carry

    lax.fori_loop(0, _NIT // 2 - 1, outer, None)
    fire(_NIT - 1, 1)
    drain(_NIT - 2, 0)
    drain(_NIT - 1, 1)


_sc_gather = functools.partial(
    pl.kernel,
    mesh=plsc.VectorSubcoreMesh(core_axis_name="c", subcore_axis_name="s"),
    out_type=jax.ShapeDtypeStruct((N,), jnp.float32),
    scratch_types=[
        pltpu.VMEM((2, _C), jnp.float32),
        pltpu.VMEM((2, _C), jnp.float32),
        pltpu.VMEM((_C,), jnp.int32),
        pltpu.VMEM((_C,), jnp.int32),
        pltpu.VMEM((_C, 8), jnp.float32),
        pltpu.VMEM((_C, 8), jnp.float32),
        pltpu.VMEM((_C,), jnp.float32),
        pltpu.SemaphoreType.DMA,
        pltpu.SemaphoreType.DMA,
    ],
    compiler_params=pltpu.CompilerParams(use_tc_tiling_on_sc=False,
                                         needs_layout_passes=False),
)(_gather_body)


def kernel(X, pixel_energy):
    e = pixel_energy.reshape(-1)
    t = _sc_build(e)
    out = _sc_gather(X[:, 0], X[:, 1], t)
    return out[:, None]
